# baseline (speedup 1.0000x reference)
import jax
import jax.numpy as jnp
from jax.experimental import pallas as pl
from jax.experimental.pallas import tpu as pltpu

_NBINS = 313
_K = 5
_INV_2SIG2 = 1.0 / 50.0
_P = 3584
_IMAX = 0x7FFFFFFF


def _encode_kernel(x_ref, q_ref, o_ref):
    pts = x_ref[0]
    q = q_ref[...]

    q2_bf = (2.0 * q).astype(jnp.bfloat16)
    p_bf = pts.astype(jnp.bfloat16)
    qp2 = jnp.dot(q2_bf, p_bf, preferred_element_type=jnp.float32)

    qf = q.astype(jnp.float32)
    q_sq = (qf[:, 0:1] * qf[:, 0:1] + qf[:, 1:2] * qf[:, 1:2])
    p_sq = (pts[0:1] * pts[0:1] + pts[1:2] * pts[1:2])
    d2 = jnp.maximum((q_sq + p_sq) - qp2, 0.0)

    iota = jax.lax.broadcasted_iota(jnp.int32, d2.shape, 0)
    keys0 = (jax.lax.bitcast_convert_type(d2, jnp.int32) & ~511) | iota
    keys = keys0
    z = jnp.zeros((1, _P), jnp.float32)
    for _ in range(_K):
        mi = jnp.min(keys, axis=0, keepdims=True)
        d2min = jax.lax.bitcast_convert_type(mi & ~511, jnp.float32)
        z = z + jnp.exp(d2min * (-_INV_2SIG2))
        keys = jnp.where(keys == mi, _IMAX, keys)
    w = jnp.exp(jax.lax.bitcast_convert_type(keys0 & ~511, jnp.float32)
                * (-_INV_2SIG2))
    o_ref[0] = jnp.where(keys == keys0, 0.0, w * (1.0 / z))


def kernel(gt_ab, q_ab):
    bs, _, H, W = gt_ab.shape
    hw = H * W
    x = gt_ab.reshape(bs, 2, hw)
    grid = (bs, hw // _P)
    out = pl.pallas_call(
        _encode_kernel,
        grid=grid,
        in_specs=[
            pl.BlockSpec((1, 2, _P), lambda i, j: (i, 0, j)),
            pl.BlockSpec((_NBINS, 2), lambda i, j: (0, 0)),
        ],
        out_specs=pl.BlockSpec((1, _NBINS, _P), lambda i, j: (i, 0, j)),
        out_shape=jax.ShapeDtypeStruct((bs, _NBINS, hw), gt_ab.dtype),
        compiler_params=pltpu.CompilerParams(
            dimension_semantics=("parallel", "parallel")),
    )(x, q_ab)
    return out.reshape(bs, _NBINS, H, W)


# speedup vs baseline: 6.4575x; 6.4575x over previous
import jax
import jax.numpy as jnp
from jax.experimental import pallas as pl
from jax.experimental.pallas import tpu as pltpu

_NBINS = 313
_K = 5
_INV_2SIG2 = 1.0 / 50.0
_ROWS = 8
_IMAX = 0x7FFFFFFF


def _bf16_round(x):
    return x.astype(jnp.bfloat16).astype(jnp.float32)


def _encode_kernel(x_ref, q_ref, o_ref):
    a = x_ref[0, 0]
    b = x_ref[0, 1]
    q = q_ref[...]
    qa = q[:, 0:1].reshape(_NBINS, 1, 1)
    qb = q[:, 1:2].reshape(_NBINS, 1, 1)

    q_sq = qa * qa + qb * qb
    p_sq = (a * a + b * b)[None]
    qp2 = (_bf16_round(2.0 * qa) * _bf16_round(a)[None]
           + _bf16_round(2.0 * qb) * _bf16_round(b)[None])
    d2 = jnp.maximum((q_sq + p_sq) - qp2, 0.0)

    iota = jax.lax.broadcasted_iota(jnp.int32, d2.shape, 0)
    keys0 = (jax.lax.bitcast_convert_type(d2, jnp.int32) & ~511) | iota
    keys = keys0
    z = jnp.zeros((1,) + p_sq.shape[1:], jnp.float32)
    for _ in range(_K):
        mi = jnp.min(keys, axis=0, keepdims=True)
        z = z + jnp.exp(jax.lax.bitcast_convert_type(mi, jnp.float32)
                        * (-_INV_2SIG2))
        keys = jnp.where(keys == mi, _IMAX, keys)
    w = jnp.exp(jax.lax.bitcast_convert_type(keys0, jnp.float32)
                * (-_INV_2SIG2))
    o_ref[0] = jnp.where(keys == keys0, 0.0, w * (1.0 / z))


def kernel(gt_ab, q_ab):
    bs, _, H, W = gt_ab.shape
    grid = (bs, H // _ROWS)
    return pl.pallas_call(
        _encode_kernel,
        grid=grid,
        in_specs=[
            pl.BlockSpec((1, 2, _ROWS, W), lambda i, j: (i, 0, j, 0)),
            pl.BlockSpec((_NBINS, 2), lambda i, j: (0, 0)),
        ],
        out_specs=pl.BlockSpec((1, _NBINS, _ROWS, W), lambda i, j: (i, 0, j, 0)),
        out_shape=jax.ShapeDtypeStruct((bs, _NBINS, H, W), gt_ab.dtype),
        compiler_params=pltpu.CompilerParams(
            dimension_semantics=("parallel", "parallel")),
    )(gt_ab, q_ab)


# speedup vs baseline: 7.1640x; 1.1094x over previous
import jax
import jax.numpy as jnp
from jax.experimental import pallas as pl
from jax.experimental.pallas import tpu as pltpu

_NBINS = 313
_K = 5
_INV_2SIG2 = 1.0 / 50.0
_ROWS = 16
_IMAX = 0x7FFFFFFF


def _bf16_round(x):
    return x.astype(jnp.bfloat16).astype(jnp.float32)


def _encode_kernel(x_ref, q_ref, o_ref):
    a = x_ref[0, 0]
    b = x_ref[0, 1]
    q = q_ref[...]
    qa = q[:, 0:1].reshape(_NBINS, 1, 1)
    qb = q[:, 1:2].reshape(_NBINS, 1, 1)

    q_sq = qa * qa + qb * qb
    p_sq = (a * a + b * b)[None]
    qp2 = (_bf16_round(2.0 * qa) * _bf16_round(a)[None]
           + _bf16_round(2.0 * qb) * _bf16_round(b)[None])
    d2 = jnp.maximum((q_sq + p_sq) - qp2, 0.0)

    iota = jax.lax.broadcasted_iota(jnp.int32, d2.shape, 0)
    keys0 = (jax.lax.bitcast_convert_type(d2, jnp.int32) & ~511) | iota
    keys = keys0
    z = jnp.zeros((1,) + p_sq.shape[1:], jnp.float32)
    for _ in range(_K):
        mi = jnp.min(keys, axis=0, keepdims=True)
        z = z + jnp.exp(jax.lax.bitcast_convert_type(mi, jnp.float32)
                        * (-_INV_2SIG2))
        keys = jnp.where(keys == mi, _IMAX, keys)
    w = jnp.exp(jax.lax.bitcast_convert_type(keys0, jnp.float32)
                * (-_INV_2SIG2))
    o_ref[0] = jnp.where(keys == keys0, 0.0, w * (1.0 / z))


def kernel(gt_ab, q_ab):
    bs, _, H, W = gt_ab.shape
    grid = (bs, H // _ROWS)
    return pl.pallas_call(
        _encode_kernel,
        grid=grid,
        in_specs=[
            pl.BlockSpec((1, 2, _ROWS, W), lambda i, j: (i, 0, j, 0)),
            pl.BlockSpec((_NBINS, 2), lambda i, j: (0, 0)),
        ],
        out_specs=pl.BlockSpec((1, _NBINS, _ROWS, W), lambda i, j: (i, 0, j, 0)),
        out_shape=jax.ShapeDtypeStruct((bs, _NBINS, H, W), gt_ab.dtype),
        compiler_params=pltpu.CompilerParams(
            dimension_semantics=("parallel", "parallel")),
    )(gt_ab, q_ab)


# speedup vs baseline: 7.1845x; 1.0029x over previous
import jax
import jax.numpy as jnp
from jax.experimental import pallas as pl
from jax.experimental.pallas import tpu as pltpu

_NBINS = 313
_K = 5
_INV_2SIG2 = 1.0 / 50.0
_ROWS = 32
_IMAX = 0x7FFFFFFF


def _bf16_round(x):
    return x.astype(jnp.bfloat16).astype(jnp.float32)


def _encode_kernel(x_ref, q_ref, o_ref):
    a = x_ref[0, 0]
    b = x_ref[0, 1]
    q = q_ref[...]
    qa = q[:, 0:1].reshape(_NBINS, 1, 1)
    qb = q[:, 1:2].reshape(_NBINS, 1, 1)

    q_sq = qa * qa + qb * qb
    p_sq = (a * a + b * b)[None]
    qp2 = (_bf16_round(2.0 * qa) * _bf16_round(a)[None]
           + _bf16_round(2.0 * qb) * _bf16_round(b)[None])
    d2 = jnp.maximum((q_sq + p_sq) - qp2, 0.0)

    iota = jax.lax.broadcasted_iota(jnp.int32, d2.shape, 0)
    keys0 = (jax.lax.bitcast_convert_type(d2, jnp.int32) & ~511) | iota
    keys = keys0
    z = jnp.zeros((1,) + p_sq.shape[1:], jnp.float32)
    for _ in range(_K):
        mi = jnp.min(keys, axis=0, keepdims=True)
        z = z + jnp.exp(jax.lax.bitcast_convert_type(mi, jnp.float32)
                        * (-_INV_2SIG2))
        keys = jnp.where(keys == mi, _IMAX, keys)
    w = jnp.exp(jax.lax.bitcast_convert_type(keys0, jnp.float32)
                * (-_INV_2SIG2))
    o_ref[0] = jnp.where(keys == keys0, 0.0, w * (1.0 / z))


def kernel(gt_ab, q_ab):
    bs, _, H, W = gt_ab.shape
    grid = (bs, H // _ROWS)
    return pl.pallas_call(
        _encode_kernel,
        grid=grid,
        in_specs=[
            pl.BlockSpec((1, 2, _ROWS, W), lambda i, j: (i, 0, j, 0)),
            pl.BlockSpec((_NBINS, 2), lambda i, j: (0, 0)),
        ],
        out_specs=pl.BlockSpec((1, _NBINS, _ROWS, W), lambda i, j: (i, 0, j, 0)),
        out_shape=jax.ShapeDtypeStruct((bs, _NBINS, H, W), gt_ab.dtype),
        compiler_params=pltpu.CompilerParams(
            dimension_semantics=("parallel", "parallel")),
    )(gt_ab, q_ab)


# speedup vs baseline: 7.9992x; 1.1134x over previous
import jax
import jax.numpy as jnp
from jax.experimental import pallas as pl
from jax.experimental.pallas import tpu as pltpu

_NBINS = 313
_K = 5
_INV_2SIG2 = 1.0 / 50.0
_ROWS = 32
_IMAX = 0x7FFFFFFF


def _bf16_round(x):
    return x.astype(jnp.bfloat16).astype(jnp.float32)


def _encode_kernel(x_ref, q_ref, o_ref):
    a = x_ref[0, 0]
    b = x_ref[0, 1]
    q = q_ref[...]
    qa = q[:, 0:1].reshape(_NBINS, 1, 1)
    qb = q[:, 1:2].reshape(_NBINS, 1, 1)

    q_sq = qa * qa + qb * qb
    p_sq = (a * a + b * b)[None]
    qp2 = (_bf16_round(2.0 * qa) * _bf16_round(a)[None]
           + _bf16_round(2.0 * qb) * _bf16_round(b)[None])
    d2 = jnp.maximum((q_sq + p_sq) - qp2, 0.0)

    iota = jax.lax.broadcasted_iota(jnp.int32, d2.shape, 0)
    keys0 = (jax.lax.bitcast_convert_type(d2, jnp.int32) & ~511) | iota
    keysb0 = jax.lax.bitcast_convert_type(keys0 + 0x00800000, jnp.float32)
    keys = keysb0
    z = jnp.zeros((1,) + p_sq.shape[1:], jnp.float32)
    for _ in range(_K):
        mi = jnp.min(keys, axis=0, keepdims=True)
        d2min = jax.lax.bitcast_convert_type(
            jax.lax.bitcast_convert_type(mi, jnp.int32) - 0x00800000,
            jnp.float32)
        z = z + jnp.exp(d2min * (-_INV_2SIG2))
        keys = jnp.where(keys == mi, jnp.inf, keys)
    w = jnp.exp(jax.lax.bitcast_convert_type(keys0, jnp.float32)
                * (-_INV_2SIG2))
    o_ref[0] = jnp.where(keys == keysb0, 0.0, w * (1.0 / z))


def kernel(gt_ab, q_ab):
    bs, _, H, W = gt_ab.shape
    grid = (bs, H // _ROWS)
    return pl.pallas_call(
        _encode_kernel,
        grid=grid,
        in_specs=[
            pl.BlockSpec((1, 2, _ROWS, W), lambda i, j: (i, 0, j, 0)),
            pl.BlockSpec((_NBINS, 2), lambda i, j: (0, 0)),
        ],
        out_specs=pl.BlockSpec((1, _NBINS, _ROWS, W), lambda i, j: (i, 0, j, 0)),
        out_shape=jax.ShapeDtypeStruct((bs, _NBINS, H, W), gt_ab.dtype),
        compiler_params=pltpu.CompilerParams(
            dimension_semantics=("parallel", "parallel")),
    )(gt_ab, q_ab)
